# Initial kernel scaffold; baseline (speedup 1.0000x reference)
import jax
import jax.numpy as jnp
from jax.experimental import pallas as pl


def kernel(z, pos, batch, embed, Wp, Wo):
    raise NotImplementedError("write your pallas kernel here")


# speedup vs baseline: 1.9650x; 1.9650x over previous
import functools

import jax
import jax.numpy as jnp
from jax import lax
from jax.experimental import pallas as pl
from jax.experimental.pallas import tpu as pltpu
from jax.experimental.pallas import tpu_sc as plsc

_H = 128
_NMOL = 1024
_N_PAD = 114688
_BLK = 7168
_ROWS = _N_PAD // 128
_W = 16
_RPW = _ROWS // _W


def _tc_body(z_ref, pos_ref, tab_ref, wp_ref, wo_ref, y_ref):
    zc = z_ref[...]
    col = lax.broadcasted_iota(jnp.int32, (_BLK, _H), 1)
    oh = (zc == col).astype(jnp.float32)
    x = jnp.dot(oh, tab_ref[...], preferred_element_type=jnp.float32)
    x = x + jnp.dot(pos_ref[...], wp_ref[...],
                    preferred_element_type=jnp.float32)
    y_ref[...] = jnp.dot(jnp.tanh(x), wo_ref[...],
                         preferred_element_type=jnp.float32)


def _tc_stage(z2, pos8, tab, wp8, wo):
    grid = (_N_PAD // _BLK,)
    return pl.pallas_call(
        _tc_body,
        grid=grid,
        in_specs=[
            pl.BlockSpec((_BLK, 1), lambda i: (i, 0)),
            pl.BlockSpec((_BLK, 8), lambda i: (i, 0)),
            pl.BlockSpec((_H, _H), lambda i: (0, 0)),
            pl.BlockSpec((8, _H), lambda i: (0, 0)),
            pl.BlockSpec((_H, 1), lambda i: (0, 0)),
        ],
        out_specs=pl.BlockSpec((_BLK, 1), lambda i: (i, 0)),
        out_shape=jax.ShapeDtypeStruct((_N_PAD, 1), jnp.float32),
    )(z2, pos8, tab, wp8, wo)


@functools.lru_cache(maxsize=None)
def _sc_scatter():
    @functools.partial(
        pl.kernel,
        out_type=jax.ShapeDtypeStruct((_NMOL,), jnp.float32),
        mesh=plsc.VectorSubcoreMesh(
            core_axis_name="c", subcore_axis_name="s",
            num_cores=1, num_subcores=_W),
        scratch_types=[
            pltpu.VMEM((_RPW, 128), jnp.float32),
            pltpu.VMEM((_RPW, 128), jnp.int32),
            pltpu.VMEM((_NMOL // _W,), jnp.float32),
            pltpu.VMEM_SHARED((_NMOL,), jnp.float32),
            pltpu.SemaphoreType.DMA,
        ],
    )
    def body(y_hbm, b_hbm, out_hbm, y_v, idx_v, z_v, acc_sh, sem):
        wid = lax.axis_index("s")
        base = wid * _RPW
        pltpu.sync_copy(y_hbm.at[pl.ds(base, _RPW)], y_v)
        pltpu.sync_copy(b_hbm.at[pl.ds(base, _RPW)], idx_v)

        zpw = _NMOL // _W
        def _zstore(i, carry):
            z_v[pl.ds(i * 16, 16)] = jnp.zeros((16,), jnp.float32)
            return carry
        lax.fori_loop(0, zpw // 16, _zstore, 0)
        pltpu.sync_copy(z_v, acc_sh.at[pl.ds(wid * zpw, zpw)])
        plsc.subcore_barrier()

        descs = [
            pltpu.async_copy(y_v.at[j], acc_sh.at[idx_v.at[j]], sem,
                             add=True)
            for j in range(_RPW)
        ]
        for d in descs:
            d.wait()
        plsc.subcore_barrier()

        @pl.when(wid == 0)
        def _():
            pltpu.sync_copy(acc_sh, out_hbm)

    return body


def kernel(z, pos, batch, embed, Wp, Wo):
    n = z.shape[0]
    pad = _N_PAD - n
    z2 = jnp.pad(z, (0, pad), constant_values=127).reshape(_N_PAD, 1)
    pos8 = jnp.pad(pos, ((0, pad), (0, 5)))
    b1 = jnp.pad(batch, (0, pad))
    tab = jnp.pad(embed, ((0, _H - embed.shape[0]), (0, 0)))
    wp8 = jnp.pad(Wp, ((0, 5), (0, 0)))

    y = _tc_stage(z2, pos8, tab, wp8, Wo)
    out = _sc_scatter()(y.reshape(_ROWS, 128), b1.reshape(_ROWS, 128))
    return out.reshape(_NMOL, 1)


# speedup vs baseline: 2.0059x; 1.0208x over previous
import functools

import jax
import jax.numpy as jnp
from jax import lax
from jax.experimental import pallas as pl
from jax.experimental.pallas import tpu as pltpu
from jax.experimental.pallas import tpu_sc as plsc

_H = 128
_NMOL = 1024
_N_PAD = 114688
_BLK = 7168
_ROWS = _N_PAD // 128
_W = 16
_RPW = _ROWS // _W


def _tc_body(z_ref, pos_ref, tab_ref, wp_ref, wo_ref, y_ref):
    zc = z_ref[...]
    col = lax.broadcasted_iota(jnp.int32, (_BLK, _H), 1)
    oh = (zc == col).astype(jnp.float32)
    x = jnp.dot(oh, tab_ref[...], preferred_element_type=jnp.float32)
    x = x + jnp.dot(pos_ref[...], wp_ref[...],
                    preferred_element_type=jnp.float32)
    y_ref[...] = jnp.dot(jnp.tanh(x), wo_ref[...],
                         preferred_element_type=jnp.float32)


def _tc_stage(z2, pos8, tab, wp8, wo):
    grid = (_N_PAD // _BLK,)
    return pl.pallas_call(
        _tc_body,
        grid=grid,
        in_specs=[
            pl.BlockSpec((_BLK, 1), lambda i: (i, 0)),
            pl.BlockSpec((_BLK, 8), lambda i: (i, 0)),
            pl.BlockSpec((_H, _H), lambda i: (0, 0)),
            pl.BlockSpec((8, _H), lambda i: (0, 0)),
            pl.BlockSpec((_H, 1), lambda i: (0, 0)),
        ],
        out_specs=pl.BlockSpec((_BLK, 1), lambda i: (i, 0)),
        out_shape=jax.ShapeDtypeStruct((_N_PAD, 1), jnp.float32),
    )(z2, pos8, tab, wp8, wo)


@functools.lru_cache(maxsize=None)
def _sc_scatter():
    npw = 128

    @functools.partial(
        pl.kernel,
        out_type=jax.ShapeDtypeStruct((_NMOL,), jnp.float32),
        mesh=plsc.VectorSubcoreMesh(
            core_axis_name="c", subcore_axis_name="s",
            num_cores=1, num_subcores=_W),
        scratch_types=[
            pltpu.VMEM((_RPW, 128), jnp.float32),
            pltpu.VMEM((_RPW, 128), jnp.int32),
            pltpu.VMEM((16 * _NMOL,), jnp.float32),
            pltpu.VMEM((_NMOL,), jnp.float32),
            pltpu.VMEM((16, npw), jnp.float32),
            pltpu.VMEM((npw,), jnp.float32),
            pltpu.VMEM_SHARED((_W, _NMOL), jnp.float32),
        ],
        compiler_params=pltpu.CompilerParams(needs_layout_passes=False),
    )
    def body(y_hbm, b_hbm, out_hbm, y_v, idx_v, acc, red, colblk, fin, stage):
        wid = lax.axis_index("s")
        base = wid * _RPW
        pltpu.sync_copy(y_hbm.at[pl.ds(base, _RPW)], y_v)
        pltpu.sync_copy(b_hbm.at[pl.ds(base, _RPW)], idx_v)

        zero16 = jnp.zeros((16,), jnp.float32)

        def _zstore(i, c):
            acc[pl.ds(i * 16, 16)] = zero16
            return c
        lax.fori_loop(0, _NMOL, _zstore, 0)

        laneoff = lax.iota(jnp.int32, 16) * _NMOL
        full = jnp.ones((16,), jnp.bool_)

        def _row(r, c):
            for cc in range(8):
                ids = idx_v[r, pl.ds(cc * 16, 16)]
                vals = y_v[r, pl.ds(cc * 16, 16)]
                plsc.addupdate_scatter(acc, [ids + laneoff], vals,
                                       mask=full)
            return c
        lax.fori_loop(0, _RPW, _row, 0)

        def _lred(j, c):
            s = acc[pl.ds(j * 16, 16)]
            for l in range(1, 16):
                s = s + acc[pl.ds(l * _NMOL + j * 16, 16)]
            red[pl.ds(j * 16, 16)] = s
            return c
        lax.fori_loop(0, _NMOL // 16, _lred, 0)

        pltpu.sync_copy(red, stage.at[wid])
        plsc.subcore_barrier()

        @pl.when(wid < _NMOL // npw)
        def _():
            pltpu.sync_copy(stage.at[:, pl.ds(wid * npw, npw)], colblk)
            for k in range(npw // 16):
                s = colblk[0, pl.ds(k * 16, 16)]
                for r in range(1, 16):
                    s = s + colblk[r, pl.ds(k * 16, 16)]
                fin[pl.ds(k * 16, 16)] = s
            pltpu.sync_copy(fin, out_hbm.at[pl.ds(wid * npw, npw)])

    return body


def kernel(z, pos, batch, embed, Wp, Wo):
    n = z.shape[0]
    pad = _N_PAD - n
    z2 = jnp.pad(z, (0, pad), constant_values=127).reshape(_N_PAD, 1)
    pos8 = jnp.pad(pos, ((0, pad), (0, 5)))
    b1 = jnp.pad(batch, (0, pad))
    tab = jnp.pad(embed, ((0, _H - embed.shape[0]), (0, 0)))
    wp8 = jnp.pad(Wp, ((0, 5), (0, 0)))

    y = _tc_stage(z2, pos8, tab, wp8, Wo)
    out = _sc_scatter()(y.reshape(_ROWS, 128), b1.reshape(_ROWS, 128))
    return out.reshape(_NMOL, 1)


# speedup vs baseline: 6.7127x; 3.3465x over previous
import functools

import jax
import jax.numpy as jnp
from jax import lax
from jax.experimental import pallas as pl
from jax.experimental.pallas import tpu as pltpu
from jax.experimental.pallas import tpu_sc as plsc

_H = 128
_NMOL = 1024
_N_PAD = 114688
_BLK = 7168
_ROWS = _N_PAD // 128
_W = 16
_RPW = _ROWS // _W


def _tc_body(z_ref, posT_ref, tabT_ref, wpT_ref, wo_ref, y_ref):
    zb = z_ref[0]
    row = lax.broadcasted_iota(jnp.int32, (_H, _BLK), 0)
    ohT = (zb == row).astype(jnp.float32)
    xT = jnp.dot(tabT_ref[...], ohT, preferred_element_type=jnp.float32)
    xT = xT + jnp.dot(wpT_ref[...], posT_ref[...],
                      preferred_element_type=jnp.float32)
    yT = jnp.sum(jnp.tanh(xT) * wo_ref[...], axis=0,
                 keepdims=True)
    y_ref[0] = yT


def _tc_stage(z3, posT, tabT, wpT, wo):
    grid = (_N_PAD // _BLK,)
    return pl.pallas_call(
        _tc_body,
        grid=grid,
        in_specs=[
            pl.BlockSpec((1, 1, _BLK), lambda i: (i, 0, 0)),
            pl.BlockSpec((3, _BLK), lambda i: (0, i)),
            pl.BlockSpec((_H, _H), lambda i: (0, 0)),
            pl.BlockSpec((_H, 3), lambda i: (0, 0)),
            pl.BlockSpec((_H, 1), lambda i: (0, 0)),
        ],
        out_specs=pl.BlockSpec((1, 1, _BLK), lambda i: (i, 0, 0)),
        out_shape=jax.ShapeDtypeStruct(
            (_N_PAD // _BLK, 1, _BLK), jnp.float32),
    )(z3, posT, tabT, wpT, wo)


@functools.lru_cache(maxsize=None)
def _sc_scatter():
    npw = 128

    @functools.partial(
        pl.kernel,
        out_type=jax.ShapeDtypeStruct((_NMOL,), jnp.float32),
        mesh=plsc.VectorSubcoreMesh(
            core_axis_name="c", subcore_axis_name="s",
            num_cores=1, num_subcores=_W),
        scratch_types=[
            pltpu.VMEM((_RPW, 128), jnp.float32),
            pltpu.VMEM((_RPW, 128), jnp.int32),
            pltpu.VMEM((16 * _NMOL,), jnp.float32),
            pltpu.VMEM((_NMOL,), jnp.float32),
            pltpu.VMEM((16, npw), jnp.float32),
            pltpu.VMEM((npw,), jnp.float32),
            pltpu.VMEM_SHARED((_W, _NMOL), jnp.float32),
        ],
        compiler_params=pltpu.CompilerParams(needs_layout_passes=False),
    )
    def body(y_hbm, b_hbm, out_hbm, y_v, idx_v, acc, red, colblk, fin, stage):
        wid = lax.axis_index("s")
        base = wid * _RPW
        pltpu.sync_copy(y_hbm.at[pl.ds(base, _RPW)], y_v)
        pltpu.sync_copy(b_hbm.at[pl.ds(base, _RPW)], idx_v)

        zero16 = jnp.zeros((16,), jnp.float32)

        def _zstore(i, c):
            acc[pl.ds(i * 16, 16)] = zero16
            return c
        lax.fori_loop(0, _NMOL, _zstore, 0)

        laneoff = lax.iota(jnp.int32, 16) * _NMOL
        full = jnp.ones((16,), jnp.bool_)

        def _row(r, c):
            for cc in range(8):
                ids = idx_v[r, pl.ds(cc * 16, 16)]
                vals = y_v[r, pl.ds(cc * 16, 16)]
                plsc.addupdate_scatter(acc, [ids + laneoff], vals,
                                       mask=full)
            return c
        lax.fori_loop(0, _RPW, _row, 0)

        def _lred(j, c):
            s = acc[pl.ds(j * 16, 16)]
            for l in range(1, 16):
                s = s + acc[pl.ds(l * _NMOL + j * 16, 16)]
            red[pl.ds(j * 16, 16)] = s
            return c
        lax.fori_loop(0, _NMOL // 16, _lred, 0)

        pltpu.sync_copy(red, stage.at[wid])
        plsc.subcore_barrier()

        @pl.when(wid < _NMOL // npw)
        def _():
            pltpu.sync_copy(stage.at[:, pl.ds(wid * npw, npw)], colblk)
            for k in range(npw // 16):
                s = colblk[0, pl.ds(k * 16, 16)]
                for r in range(1, 16):
                    s = s + colblk[r, pl.ds(k * 16, 16)]
                fin[pl.ds(k * 16, 16)] = s
            pltpu.sync_copy(fin, out_hbm.at[pl.ds(wid * npw, npw)])

    return body


def kernel(z, pos, batch, embed, Wp, Wo):
    n = z.shape[0]
    pad = _N_PAD - n
    z3 = jnp.pad(z, (0, pad), constant_values=127).reshape(
        _N_PAD // _BLK, 1, _BLK)
    posT = jnp.pad(pos, ((0, pad), (0, 0))).T
    b1 = jnp.pad(batch, (0, pad))
    tabT = jnp.pad(embed, ((0, _H - embed.shape[0]), (0, 0))).T
    wpT = Wp.T

    y = _tc_stage(z3, posT, tabT, wpT, Wo)
    out = _sc_scatter()(y.reshape(_ROWS, 128), b1.reshape(_ROWS, 128))
    return out.reshape(_NMOL, 1)


# speedup vs baseline: 8.1620x; 1.2159x over previous
import functools

import jax
import jax.numpy as jnp
from jax import lax
from jax.experimental import pallas as pl
from jax.experimental.pallas import tpu as pltpu
from jax.experimental.pallas import tpu_sc as plsc

_H = 128
_NMOL = 1024
_N_PAD = 114688
_BLK = 7168
_ROWS = _N_PAD // 128
_W = 16
_RPW = _ROWS // _W


def _tc_body(z_ref, posT_ref, tabT_ref, wo_ref, y_ref):
    zb = z_ref[0]
    row = lax.broadcasted_iota(jnp.int32, (_H, _BLK), 0)
    ohT = (zb == row).astype(jnp.bfloat16)
    rhs = jnp.concatenate(
        [ohT, posT_ref[...].astype(jnp.bfloat16)], axis=0)
    xT = jnp.dot(tabT_ref[...], rhs, preferred_element_type=jnp.float32)
    yT = jnp.sum(jnp.tanh(xT) * wo_ref[...], axis=0,
                 keepdims=True)
    y_ref[0] = yT


def _tc_stage(z3, posT8, tabT2, wo):
    grid = (_N_PAD // _BLK,)
    return pl.pallas_call(
        _tc_body,
        grid=grid,
        in_specs=[
            pl.BlockSpec((1, 1, _BLK), lambda i: (i, 0, 0)),
            pl.BlockSpec((8, _BLK), lambda i: (0, i)),
            pl.BlockSpec((_H, _H + 8), lambda i: (0, 0)),
            pl.BlockSpec((_H, 1), lambda i: (0, 0)),
        ],
        out_specs=pl.BlockSpec((1, 1, _BLK), lambda i: (i, 0, 0)),
        out_shape=jax.ShapeDtypeStruct(
            (_N_PAD // _BLK, 1, _BLK), jnp.float32),
    )(z3, posT8, tabT2, wo)


@functools.lru_cache(maxsize=None)
def _sc_scatter():
    npw = 128

    @functools.partial(
        pl.kernel,
        out_type=jax.ShapeDtypeStruct((_NMOL,), jnp.float32),
        mesh=plsc.VectorSubcoreMesh(
            core_axis_name="c", subcore_axis_name="s",
            num_cores=1, num_subcores=_W),
        scratch_types=[
            pltpu.VMEM((_RPW, 128), jnp.float32),
            pltpu.VMEM((_RPW, 128), jnp.int32),
            pltpu.VMEM((16 * _NMOL,), jnp.float32),
            pltpu.VMEM((_NMOL,), jnp.float32),
            pltpu.VMEM((16, npw), jnp.float32),
            pltpu.VMEM((npw,), jnp.float32),
            pltpu.VMEM_SHARED((_W, _NMOL), jnp.float32),
        ],
        compiler_params=pltpu.CompilerParams(needs_layout_passes=False),
    )
    def body(y_hbm, b_hbm, out_hbm, y_v, idx_v, acc, red, colblk, fin, stage):
        wid = lax.axis_index("s")
        base = wid * _RPW
        pltpu.sync_copy(y_hbm.at[pl.ds(base, _RPW)], y_v)
        pltpu.sync_copy(b_hbm.at[pl.ds(base, _RPW)], idx_v)

        zero16 = jnp.zeros((16,), jnp.float32)

        def _zstore(i, c):
            for u in range(16):
                acc[pl.ds(i * 256 + u * 16, 16)] = zero16
            return c
        lax.fori_loop(0, _NMOL // 16, _zstore, 0)

        laneoff = lax.iota(jnp.int32, 16) * _NMOL
        full = jnp.ones((16,), jnp.bool_)

        def _row(r, c):
            for cc in range(8):
                ids = idx_v[r, pl.ds(cc * 16, 16)]
                vals = y_v[r, pl.ds(cc * 16, 16)]
                plsc.addupdate_scatter(acc, [ids + laneoff], vals,
                                       mask=full)
            return c
        lax.fori_loop(0, _RPW, _row, 0)

        def _lred(j, c):
            s = acc[pl.ds(j * 16, 16)]
            for l in range(1, 16):
                s = s + acc[pl.ds(l * _NMOL + j * 16, 16)]
            red[pl.ds(j * 16, 16)] = s
            return c
        lax.fori_loop(0, _NMOL // 16, _lred, 0)

        pltpu.sync_copy(red, stage.at[wid])
        plsc.subcore_barrier()

        @pl.when(wid < _NMOL // npw)
        def _():
            pltpu.sync_copy(stage.at[:, pl.ds(wid * npw, npw)], colblk)
            for k in range(npw // 16):
                s = colblk[0, pl.ds(k * 16, 16)]
                for r in range(1, 16):
                    s = s + colblk[r, pl.ds(k * 16, 16)]
                fin[pl.ds(k * 16, 16)] = s
            pltpu.sync_copy(fin, out_hbm.at[pl.ds(wid * npw, npw)])

    return body


def kernel(z, pos, batch, embed, Wp, Wo):
    n = z.shape[0]
    pad = _N_PAD - n
    z3 = jnp.pad(z, (0, pad), constant_values=127).reshape(
        _N_PAD // _BLK, 1, _BLK)
    posT8 = jnp.pad(pos, ((0, pad), (0, 5))).T
    b1 = jnp.pad(batch, (0, pad))
    tabT2 = jnp.concatenate(
        [jnp.pad(embed, ((0, _H - embed.shape[0]), (0, 0))).T,
         Wp.T, jnp.zeros((_H, 5), jnp.float32)],
        axis=1).astype(jnp.bfloat16)

    y = _tc_stage(z3, posT8, tabT2, Wo)
    out = _sc_scatter()(y.reshape(_ROWS, 128), b1.reshape(_ROWS, 128))
    return out.reshape(_NMOL, 1)


# speedup vs baseline: 8.9587x; 1.0976x over previous
import functools

import jax
import jax.numpy as jnp
from jax import lax
from jax.experimental import pallas as pl
from jax.experimental.pallas import tpu as pltpu
from jax.experimental.pallas import tpu_sc as plsc

_H = 128
_NMOL = 1024
_N_PAD = 114688
_BLK = 14336
_ROWS = _N_PAD // 128
_W = 16
_RPW = _ROWS // _W


def _tc_body(z_ref, posT_ref, tabT_ref, wo_ref, y_ref):
    zb = z_ref[0]
    row = lax.broadcasted_iota(jnp.int32, (_H, _BLK), 0)
    ohT = (zb == row).astype(jnp.bfloat16)
    rhs = jnp.concatenate([ohT, posT_ref[...]], axis=0)
    xT = jnp.dot(tabT_ref[...], rhs, preferred_element_type=jnp.float32)
    yT = jnp.sum(jnp.tanh(xT) * wo_ref[...], axis=0,
                 keepdims=True)
    y_ref[0] = yT


def _tc_stage(z3, posT8, tabT2, wo):
    grid = (_N_PAD // _BLK,)
    return pl.pallas_call(
        _tc_body,
        grid=grid,
        in_specs=[
            pl.BlockSpec((1, 1, _BLK), lambda i: (i, 0, 0)),
            pl.BlockSpec((8, _BLK), lambda i: (0, i)),
            pl.BlockSpec((_H, _H + 8), lambda i: (0, 0)),
            pl.BlockSpec((_H, 1), lambda i: (0, 0)),
        ],
        out_specs=pl.BlockSpec((1, 1, _BLK), lambda i: (i, 0, 0)),
        out_shape=jax.ShapeDtypeStruct(
            (_N_PAD // _BLK, 1, _BLK), jnp.float32),
    )(z3, posT8, tabT2, wo)


@functools.lru_cache(maxsize=None)
def _sc_scatter():
    npw = 128

    @functools.partial(
        pl.kernel,
        out_type=jax.ShapeDtypeStruct((_NMOL,), jnp.float32),
        mesh=plsc.VectorSubcoreMesh(
            core_axis_name="c", subcore_axis_name="s",
            num_cores=1, num_subcores=_W),
        scratch_types=[
            pltpu.VMEM((_RPW, 128), jnp.float32),
            pltpu.VMEM((_RPW, 128), jnp.int32),
            pltpu.VMEM((16 * _NMOL,), jnp.float32),
            pltpu.VMEM((_NMOL,), jnp.float32),
            pltpu.VMEM((16, npw), jnp.float32),
            pltpu.VMEM((npw,), jnp.float32),
            pltpu.VMEM_SHARED((_W, _NMOL), jnp.float32),
            pltpu.SemaphoreType.DMA,
        ],
        compiler_params=pltpu.CompilerParams(needs_layout_passes=False),
    )
    def body(y_hbm, b_hbm, out_hbm, y_v, idx_v, acc, red, colblk, fin,
             stage, sem):
        wid = lax.axis_index("s")
        base = wid * _RPW
        d1 = pltpu.async_copy(y_hbm.at[pl.ds(base, _RPW)], y_v, sem)
        d2 = pltpu.async_copy(b_hbm.at[pl.ds(base, _RPW)], idx_v, sem)

        zero16 = jnp.zeros((16,), jnp.float32)

        def _zstore(i, c):
            for u in range(16):
                acc[pl.ds(i * 256 + u * 16, 16)] = zero16
            return c
        lax.fori_loop(0, _NMOL // 16, _zstore, 0)
        d1.wait()
        d2.wait()

        laneoff = lax.iota(jnp.int32, 16) * _NMOL
        full = jnp.ones((16,), jnp.bool_)

        def _row(r, c):
            for cc in range(8):
                ids = idx_v[r, pl.ds(cc * 16, 16)]
                vals = y_v[r, pl.ds(cc * 16, 16)]
                plsc.addupdate_scatter(acc, [ids + laneoff], vals,
                                       mask=full)
            return c
        lax.fori_loop(0, _RPW, _row, 0)

        def _lred(j, c):
            s = acc[pl.ds(j * 16, 16)]
            for l in range(1, 16):
                s = s + acc[pl.ds(l * _NMOL + j * 16, 16)]
            red[pl.ds(j * 16, 16)] = s
            return c
        lax.fori_loop(0, _NMOL // 16, _lred, 0)

        pltpu.sync_copy(red, stage.at[wid])
        plsc.subcore_barrier()

        @pl.when(wid < _NMOL // npw)
        def _():
            pltpu.sync_copy(stage.at[:, pl.ds(wid * npw, npw)], colblk)
            for k in range(npw // 16):
                s = colblk[0, pl.ds(k * 16, 16)]
                for r in range(1, 16):
                    s = s + colblk[r, pl.ds(k * 16, 16)]
                fin[pl.ds(k * 16, 16)] = s
            pltpu.sync_copy(fin, out_hbm.at[pl.ds(wid * npw, npw)])

    return body


def kernel(z, pos, batch, embed, Wp, Wo):
    n = z.shape[0]
    pad = _N_PAD - n
    z3 = jnp.pad(z, (0, pad), constant_values=127).reshape(
        _N_PAD // _BLK, 1, _BLK)
    posT8 = jnp.pad(pos, ((0, pad), (0, 5))).T.astype(
        jnp.bfloat16)
    b1 = jnp.pad(batch, (0, pad))
    tabT2 = jnp.concatenate(
        [jnp.pad(embed, ((0, _H - embed.shape[0]), (0, 0))).T,
         Wp.T, jnp.zeros((_H, 5), jnp.float32)],
        axis=1).astype(jnp.bfloat16)

    y = _tc_stage(z3, posT8, tabT2, Wo)
    out = _sc_scatter()(y.reshape(_ROWS, 128), b1.reshape(_ROWS, 128))
    return out.reshape(_NMOL, 1)


# speedup vs baseline: 9.0762x; 1.0131x over previous
import functools

import jax
import jax.numpy as jnp
from jax import lax
from jax.experimental import pallas as pl
from jax.experimental.pallas import tpu as pltpu
from jax.experimental.pallas import tpu_sc as plsc

_H = 128
_NMOL = 1024
_N_PAD = 114688
_BLK = 28672
_ROWS = _N_PAD // 128
_W = 16
_RPW = _ROWS // _W


def _tc_body(z_ref, posT_ref, tabT_ref, wo_ref, y_ref):
    zb = z_ref[0]
    row = lax.broadcasted_iota(jnp.int32, (_H, _BLK), 0)
    ohT = (zb == row).astype(jnp.bfloat16)
    rhs = jnp.concatenate([ohT, posT_ref[...]], axis=0)
    xT = jnp.dot(tabT_ref[...], rhs, preferred_element_type=jnp.float32)
    yT = jnp.sum(jnp.tanh(xT) * wo_ref[...], axis=0,
                 keepdims=True)
    y_ref[0] = yT


def _tc_stage(z3, posT8, tabT2, wo):
    grid = (_N_PAD // _BLK,)
    return pl.pallas_call(
        _tc_body,
        grid=grid,
        in_specs=[
            pl.BlockSpec((1, 1, _BLK), lambda i: (i, 0, 0)),
            pl.BlockSpec((8, _BLK), lambda i: (0, i)),
            pl.BlockSpec((_H, _H + 8), lambda i: (0, 0)),
            pl.BlockSpec((_H, 1), lambda i: (0, 0)),
        ],
        out_specs=pl.BlockSpec((1, 1, _BLK), lambda i: (i, 0, 0)),
        out_shape=jax.ShapeDtypeStruct(
            (_N_PAD // _BLK, 1, _BLK), jnp.float32),
    )(z3, posT8, tabT2, wo)


@functools.lru_cache(maxsize=None)
def _sc_scatter():
    npw = 128

    @functools.partial(
        pl.kernel,
        out_type=jax.ShapeDtypeStruct((_NMOL,), jnp.float32),
        mesh=plsc.VectorSubcoreMesh(
            core_axis_name="c", subcore_axis_name="s",
            num_cores=1, num_subcores=_W),
        scratch_types=[
            pltpu.VMEM((_RPW, 128), jnp.float32),
            pltpu.VMEM((_RPW, 128), jnp.int32),
            pltpu.VMEM((16 * _NMOL,), jnp.float32),
            pltpu.VMEM((_NMOL,), jnp.float32),
            pltpu.VMEM((16, npw), jnp.float32),
            pltpu.VMEM((npw,), jnp.float32),
            pltpu.VMEM_SHARED((_W, _NMOL), jnp.float32),
            pltpu.SemaphoreType.DMA,
        ],
        compiler_params=pltpu.CompilerParams(needs_layout_passes=False),
    )
    def body(y_hbm, b_hbm, out_hbm, y_v, idx_v, acc, red, colblk, fin,
             stage, sem):
        wid = lax.axis_index("s")
        base = wid * _RPW
        d1 = pltpu.async_copy(y_hbm.at[pl.ds(base, _RPW)], y_v, sem)
        d2 = pltpu.async_copy(b_hbm.at[pl.ds(base, _RPW)], idx_v, sem)

        zero16 = jnp.zeros((16,), jnp.float32)

        def _zstore(i, c):
            for u in range(16):
                acc[pl.ds(i * 256 + u * 16, 16)] = zero16
            return c
        lax.fori_loop(0, _NMOL // 16, _zstore, 0)
        d1.wait()
        d2.wait()

        laneoff = lax.iota(jnp.int32, 16) * _NMOL
        full = jnp.ones((16,), jnp.bool_)

        def _row(r, c):
            for cc in range(8):
                ids = idx_v[r, pl.ds(cc * 16, 16)]
                vals = y_v[r, pl.ds(cc * 16, 16)]
                plsc.addupdate_scatter(acc, [ids + laneoff], vals,
                                       mask=full)
            return c
        lax.fori_loop(0, _RPW, _row, 0)

        def _lred(j, c):
            s = acc[pl.ds(j * 16, 16)]
            for l in range(1, 16):
                s = s + acc[pl.ds(l * _NMOL + j * 16, 16)]
            red[pl.ds(j * 16, 16)] = s
            return c
        lax.fori_loop(0, _NMOL // 16, _lred, 0)

        pltpu.sync_copy(red, stage.at[wid])
        plsc.subcore_barrier()

        @pl.when(wid < _NMOL // npw)
        def _():
            pltpu.sync_copy(stage.at[:, pl.ds(wid * npw, npw)], colblk)
            for k in range(npw // 16):
                s = colblk[0, pl.ds(k * 16, 16)]
                for r in range(1, 16):
                    s = s + colblk[r, pl.ds(k * 16, 16)]
                fin[pl.ds(k * 16, 16)] = s
            pltpu.sync_copy(fin, out_hbm.at[pl.ds(wid * npw, npw)])

    return body


def kernel(z, pos, batch, embed, Wp, Wo):
    n = z.shape[0]
    pad = _N_PAD - n
    z3 = jnp.pad(z, (0, pad), constant_values=127).reshape(
        _N_PAD // _BLK, 1, _BLK)
    posT8 = jnp.pad(pos.astype(jnp.bfloat16),
                    ((0, pad), (0, 5))).T
    b1 = jnp.pad(batch, (0, pad))
    tabT2 = jnp.concatenate(
        [jnp.pad(embed, ((0, _H - embed.shape[0]), (0, 0))).T,
         Wp.T, jnp.zeros((_H, 5), jnp.float32)],
        axis=1).astype(jnp.bfloat16)

    y = _tc_stage(z3, posT8, tabT2, Wo)
    out = _sc_scatter()(y.reshape(_ROWS, 128), b1.reshape(_ROWS, 128))
    return out.reshape(_NMOL, 1)


# speedup vs baseline: 9.5398x; 1.0511x over previous
import functools

import jax
import jax.numpy as jnp
from jax import lax
from jax.experimental import pallas as pl
from jax.experimental.pallas import tpu as pltpu
from jax.experimental.pallas import tpu_sc as plsc

_H = 128
_NMOL = 1024
_N_PAD = 114688
_BLK = 28672
_ROWS = _N_PAD // 128
_W = 16
_RPW = _ROWS // _W


def _tc_body(z_ref, posT_ref, tabT_ref, wo_ref, y_ref):
    zb = z_ref[0]
    row = lax.broadcasted_iota(jnp.int32, (_H, _BLK), 0)
    ohT = (zb == row).astype(jnp.bfloat16)
    rhs = jnp.concatenate([ohT, posT_ref[...]], axis=0)
    xT = jnp.dot(tabT_ref[...], rhs, preferred_element_type=jnp.float32)
    yT = jnp.sum(jnp.tanh(xT) * wo_ref[...], axis=0,
                 keepdims=True)
    y_ref[0] = yT


def _tc_stage(z3, posT8, tabT2, wo):
    grid = (_N_PAD // _BLK,)
    return pl.pallas_call(
        _tc_body,
        grid=grid,
        in_specs=[
            pl.BlockSpec((1, 1, _BLK), lambda i: (i, 0, 0)),
            pl.BlockSpec((8, _BLK), lambda i: (0, i)),
            pl.BlockSpec((_H, _H + 8), lambda i: (0, 0)),
            pl.BlockSpec((_H, 1), lambda i: (0, 0)),
        ],
        out_specs=pl.BlockSpec((1, 1, _BLK), lambda i: (i, 0, 0)),
        out_shape=jax.ShapeDtypeStruct(
            (_N_PAD // _BLK, 1, _BLK), jnp.float32),
    )(z3, posT8, tabT2, wo)


@functools.lru_cache(maxsize=None)
def _sc_scatter():
    npw = 128

    @functools.partial(
        pl.kernel,
        out_type=jax.ShapeDtypeStruct((_NMOL,), jnp.float32),
        mesh=plsc.VectorSubcoreMesh(
            core_axis_name="c", subcore_axis_name="s",
            num_cores=1, num_subcores=_W),
        scratch_types=[
            pltpu.VMEM((_RPW, 128), jnp.float32),
            pltpu.VMEM((_RPW, 128), jnp.int32),
            pltpu.VMEM((16 * _NMOL,), jnp.float32),
            pltpu.VMEM((_NMOL,), jnp.float32),
            pltpu.VMEM((16, npw), jnp.float32),
            pltpu.VMEM((npw,), jnp.float32),
            pltpu.VMEM_SHARED((_W, _NMOL), jnp.float32),
            pltpu.SemaphoreType.DMA,
        ],
        compiler_params=pltpu.CompilerParams(needs_layout_passes=False),
    )
    def body(y_hbm, b_hbm, out_hbm, y_v, idx_v, acc, red, colblk, fin,
             stage, sem):
        wid = lax.axis_index("s")
        base = wid * _RPW
        d1 = pltpu.async_copy(y_hbm.at[pl.ds(base, _RPW)], y_v, sem)
        d2 = pltpu.async_copy(b_hbm.at[pl.ds(base, _RPW)], idx_v, sem)

        zero16 = jnp.zeros((16,), jnp.float32)

        def _zstore(i, c):
            for u in range(16):
                acc[pl.ds(i * 256 + u * 16, 16)] = zero16
            return c
        lax.fori_loop(0, _NMOL // 16, _zstore, 0)
        d1.wait()
        d2.wait()

        laneoff = lax.iota(jnp.int32, 16) * _NMOL
        full = jnp.ones((16,), jnp.bool_)

        def _row(r, c):
            addrs = [idx_v[r, pl.ds(cc * 16, 16)] + laneoff
                     for cc in range(8)]
            valss = [y_v[r, pl.ds(cc * 16, 16)] for cc in range(8)]
            for cc in range(8):
                plsc.addupdate_scatter(acc, [addrs[cc]], valss[cc],
                                       mask=full)
            return c
        lax.fori_loop(0, _RPW, _row, 0)

        def _lred(j, c):
            s = acc[pl.ds(j * 16, 16)]
            for l in range(1, 16):
                s = s + acc[pl.ds(l * _NMOL + j * 16, 16)]
            red[pl.ds(j * 16, 16)] = s
            return c
        lax.fori_loop(0, _NMOL // 16, _lred, 0)

        pltpu.sync_copy(red, stage.at[wid])
        plsc.subcore_barrier()

        @pl.when(wid < _NMOL // npw)
        def _():
            pltpu.sync_copy(stage.at[:, pl.ds(wid * npw, npw)], colblk)
            for k in range(npw // 16):
                s = colblk[0, pl.ds(k * 16, 16)]
                for r in range(1, 16):
                    s = s + colblk[r, pl.ds(k * 16, 16)]
                fin[pl.ds(k * 16, 16)] = s
            pltpu.sync_copy(fin, out_hbm.at[pl.ds(wid * npw, npw)])

    return body


def kernel(z, pos, batch, embed, Wp, Wo):
    n = z.shape[0]
    pad = _N_PAD - n
    z3 = jnp.pad(z, (0, pad), constant_values=127).reshape(
        _N_PAD // _BLK, 1, _BLK)
    posT8 = jnp.pad(pos.astype(jnp.bfloat16),
                    ((0, pad), (0, 5))).T
    b1 = jnp.pad(batch, (0, pad))
    tabT2 = jnp.concatenate(
        [jnp.pad(embed, ((0, _H - embed.shape[0]), (0, 0))).T,
         Wp.T, jnp.zeros((_H, 5), jnp.float32)],
        axis=1).astype(jnp.bfloat16)

    y = _tc_stage(z3, posT8, tabT2, Wo)
    out = _sc_scatter()(y.reshape(_ROWS, 128), b1.reshape(_ROWS, 128))
    return out.reshape(_NMOL, 1)


# speedup vs baseline: 9.5601x; 1.0021x over previous
import functools

import jax
import jax.numpy as jnp
from jax import lax
from jax.experimental import pallas as pl
from jax.experimental.pallas import tpu as pltpu
from jax.experimental.pallas import tpu_sc as plsc

_H = 128
_NMOL = 1024
_N_PAD = 114688
_BLK = 57344
_ROWS = _N_PAD // 128
_W = 16
_RPW = _ROWS // _W


def _tc_body(z_ref, posT_ref, tabT_ref, wo_ref, y_ref):
    zb = z_ref[0]
    row = lax.broadcasted_iota(jnp.int32, (_H, _BLK), 0)
    ohT = (zb == row).astype(jnp.bfloat16)
    rhs = jnp.concatenate([ohT, posT_ref[...]], axis=0)
    xT = jnp.dot(tabT_ref[...], rhs, preferred_element_type=jnp.float32)
    yT = jnp.sum(jnp.tanh(xT) * wo_ref[...], axis=0,
                 keepdims=True)
    y_ref[0] = yT


def _tc_stage(z3, posT8, tabT2, wo):
    grid = (_N_PAD // _BLK,)
    return pl.pallas_call(
        _tc_body,
        grid=grid,
        in_specs=[
            pl.BlockSpec((1, 1, _BLK), lambda i: (i, 0, 0)),
            pl.BlockSpec((8, _BLK), lambda i: (0, i)),
            pl.BlockSpec((_H, _H + 8), lambda i: (0, 0)),
            pl.BlockSpec((_H, 1), lambda i: (0, 0)),
        ],
        out_specs=pl.BlockSpec((1, 1, _BLK), lambda i: (i, 0, 0)),
        out_shape=jax.ShapeDtypeStruct(
            (_N_PAD // _BLK, 1, _BLK), jnp.float32),
    )(z3, posT8, tabT2, wo)


@functools.lru_cache(maxsize=None)
def _sc_scatter():
    npw = 128

    @functools.partial(
        pl.kernel,
        out_type=jax.ShapeDtypeStruct((_NMOL,), jnp.float32),
        mesh=plsc.VectorSubcoreMesh(
            core_axis_name="c", subcore_axis_name="s",
            num_cores=1, num_subcores=_W),
        scratch_types=[
            pltpu.VMEM((_RPW, 128), jnp.float32),
            pltpu.VMEM((_RPW, 128), jnp.int32),
            pltpu.VMEM((16 * _NMOL,), jnp.float32),
            pltpu.VMEM((_NMOL,), jnp.float32),
            pltpu.VMEM((16, npw), jnp.float32),
            pltpu.VMEM((npw,), jnp.float32),
            pltpu.VMEM_SHARED((_W, _NMOL), jnp.float32),
            pltpu.SemaphoreType.DMA,
        ],
        compiler_params=pltpu.CompilerParams(needs_layout_passes=False),
    )
    def body(y_hbm, b_hbm, out_hbm, y_v, idx_v, acc, red, colblk, fin,
             stage, sem):
        wid = lax.axis_index("s")
        base = wid * _RPW
        d1 = pltpu.async_copy(y_hbm.at[pl.ds(base, _RPW)], y_v, sem)
        d2 = pltpu.async_copy(b_hbm.at[pl.ds(base, _RPW)], idx_v, sem)

        zero16 = jnp.zeros((16,), jnp.float32)

        def _zstore(i, c):
            for u in range(16):
                acc[pl.ds(i * 256 + u * 16, 16)] = zero16
            return c
        lax.fori_loop(0, _NMOL // 16, _zstore, 0)
        d1.wait()
        d2.wait()

        laneoff = lax.iota(jnp.int32, 16) * _NMOL
        full = jnp.ones((16,), jnp.bool_)

        def _row(r, c):
            addrs = [idx_v[r, pl.ds(cc * 16, 16)] + laneoff
                     for cc in range(8)]
            valss = [y_v[r, pl.ds(cc * 16, 16)] for cc in range(8)]
            for cc in range(8):
                plsc.addupdate_scatter(acc, [addrs[cc]], valss[cc],
                                       mask=full)
            return c
        lax.fori_loop(0, _RPW, _row, 0)

        def _lred(j, c):
            for u in range(2):
                o = j * 32 + u * 16
                s = acc[pl.ds(o, 16)]
                for l in range(1, 16):
                    s = s + acc[pl.ds(l * _NMOL + o, 16)]
                red[pl.ds(o, 16)] = s
            return c
        lax.fori_loop(0, _NMOL // 32, _lred, 0)

        pltpu.sync_copy(red, stage.at[wid])
        plsc.subcore_barrier()

        @pl.when(wid < _NMOL // npw)
        def _():
            pltpu.sync_copy(stage.at[:, pl.ds(wid * npw, npw)], colblk)
            for k in range(npw // 16):
                s = colblk[0, pl.ds(k * 16, 16)]
                for r in range(1, 16):
                    s = s + colblk[r, pl.ds(k * 16, 16)]
                fin[pl.ds(k * 16, 16)] = s
            pltpu.sync_copy(fin, out_hbm.at[pl.ds(wid * npw, npw)])

    return body


def kernel(z, pos, batch, embed, Wp, Wo):
    n = z.shape[0]
    pad = _N_PAD - n
    z3 = jnp.pad(z, (0, pad), constant_values=127).reshape(
        _N_PAD // _BLK, 1, _BLK)
    posT8 = jnp.pad(pos.astype(jnp.bfloat16),
                    ((0, pad), (0, 5))).T
    b1 = jnp.pad(batch, (0, pad))
    tabT2 = jnp.concatenate(
        [jnp.pad(embed, ((0, _H - embed.shape[0]), (0, 0))).T,
         Wp.T, jnp.zeros((_H, 5), jnp.float32)],
        axis=1).astype(jnp.bfloat16)

    y = _tc_stage(z3, posT8, tabT2, Wo)
    out = _sc_scatter()(y.reshape(_ROWS, 128), b1.reshape(_ROWS, 128))
    return out.reshape(_NMOL, 1)
